# baseline (speedup 1.0000x reference)
import functools

import jax
import jax.numpy as jnp
from jax import lax
from jax.experimental import pallas as pl
from jax.experimental.pallas import tpu as pltpu
from jax.experimental.pallas import tpu_sc as plsc

_B = 4096
_T = 200
_D = 64
_N = _B * _T
_L = 16
_NC = 2
_NS = 16
_NW = _NC * _NS
_PW = _N // _NW
_C = 1024
_G = 128
_CHUNKS = _PW // _C


def _make_sc_gather():
    mesh = plsc.VectorSubcoreMesh(core_axis_name="c", subcore_axis_name="s")

    @functools.partial(
        pl.kernel,
        mesh=mesh,
        out_type=jax.ShapeDtypeStruct((_N, _D), jnp.float32),
        scratch_types=[
            pltpu.VMEM((_C // _G, _G), jnp.int32),
            pltpu.VMEM((_C, _D), jnp.float32),
            pltpu.VMEM((_L,), jnp.int32),
            pltpu.SemaphoreType.DMA,
        ],
        compiler_params=pltpu.CompilerParams(use_tc_tiling_on_sc=False),
    )
    def body(text_hbm, seqv_hbm, table_hbm, out_hbm, idx_v, rows_v, seq_v, sem):
        wid = lax.axis_index("s") * _NC + lax.axis_index("c")
        pltpu.sync_copy(seqv_hbm, seq_v)
        seq = seq_v[...]
        lanes = lax.iota(jnp.int32, _L)
        base_row = wid * _PW

        def chunk_body(g, carry):
            row0 = pl.multiple_of(base_row + g * _C, _C)
            pltpu.sync_copy(
                text_hbm.at[pl.ds(pl.multiple_of(row0 // _G, _C // _G), _C // _G)],
                idx_v,
            )
            for j in range(_C // _G):
                for i in range(_G // _L):
                    off = j * _G + i * _L
                    v = idx_v[j, pl.ds(i * _L, _L)]
                    pos = lax.rem(row0 + off + lanes, _T)
                    idx_v[j, pl.ds(i * _L, _L)] = jnp.where(pos < seq, v + 1, 0)
            copies = [
                pltpu.async_copy(
                    table_hbm.at[idx_v.at[j]],
                    rows_v.at[pl.ds(j * _G, _G)],
                    sem,
                )
                for j in range(_C // _G)
            ]
            for cp in copies:
                cp.wait()
            pltpu.sync_copy(rows_v, out_hbm.at[pl.ds(row0, _C)])
            return carry

        lax.fori_loop(0, _CHUNKS, chunk_body, 0)

    return body


_sc_gather = _make_sc_gather()


def kernel(text, seq_len, text_embed):
    text2d = text.reshape(_N // _G, _G)
    seqv = jnp.full((_L,), seq_len, dtype=jnp.int32)
    out = _sc_gather(text2d, seqv, text_embed)
    return out.reshape(_B, _T, _D)


# speedup vs baseline: 1.0192x; 1.0192x over previous
import functools

import jax
import jax.numpy as jnp
from jax import lax
from jax.experimental import pallas as pl
from jax.experimental.pallas import tpu as pltpu
from jax.experimental.pallas import tpu_sc as plsc

_B = 4096
_T = 200
_D = 64
_N = _B * _T
_L = 16
_NC = 2
_NS = 16
_NW = _NC * _NS
_PW = _N // _NW
_G = 128
_C = 256
_NB = 4
_Q = _PW // _C
_QG = _C // _G
_IR = _PW // _G


def _make_sc_gather():
    mesh = plsc.VectorSubcoreMesh(core_axis_name="c", subcore_axis_name="s")

    @functools.partial(
        pl.kernel,
        mesh=mesh,
        out_type=jax.ShapeDtypeStruct((_N, _D), jnp.float32),
        scratch_types=[
            pltpu.VMEM((_IR, _G), jnp.int32),
            pltpu.VMEM((_L,), jnp.int32),
        ]
        + [pltpu.VMEM((_C, _D), jnp.float32) for _ in range(_NB)]
        + [pltpu.SemaphoreType.DMA for _ in range(2 * _NB)],
        compiler_params=pltpu.CompilerParams(use_tc_tiling_on_sc=False),
    )
    def body(text_hbm, seqv_hbm, table_hbm, out_hbm, idx_v, seq_v, *bufs):
        rows = list(bufs[:_NB])
        gsem = list(bufs[_NB:2 * _NB])
        wsem = list(bufs[2 * _NB:])
        wid = lax.axis_index("s") * _NC + lax.axis_index("c")
        base_row = wid * _PW

        pltpu.sync_copy(seqv_hbm, seq_v)
        seq = seq_v[...]
        lanes = lax.iota(jnp.int32, _L)

        pltpu.sync_copy(
            text_hbm.at[pl.ds(pl.multiple_of(base_row // _G, 8), _IR)], idx_v
        )

        def adj(r, carry):
            rbase = r * _G
            for i in range(_G // _L):
                v = idx_v[r, pl.ds(i * _L, _L)]
                pos = lax.rem(rbase + i * _L + lanes, _T)
                idx_v[r, pl.ds(i * _L, _L)] = jnp.where(pos < seq, v + 1, 0)
            return carry

        lax.fori_loop(0, _IR, adj, 0)

        def fire(q, b):
            for j in range(_QG):
                pltpu.async_copy(
                    table_hbm.at[idx_v.at[q * _QG + j]],
                    rows[b].at[pl.ds(j * _G, _G)],
                    gsem[b],
                )

        def wait_gathers(b):
            for j in range(_QG):
                pltpu.make_async_copy(
                    table_hbm.at[idx_v.at[0]],
                    rows[b].at[pl.ds(j * _G, _G)],
                    gsem[b],
                ).wait()

        def writeback(q, b):
            row0 = pl.multiple_of(base_row + q * _C, _C)
            pltpu.async_copy(rows[b], out_hbm.at[pl.ds(row0, _C)], wsem[b])

        def wait_writeback(b):
            pltpu.make_async_copy(
                rows[b],
                out_hbm.at[pl.ds(pl.multiple_of(base_row, _C), _C)],
                wsem[b],
            ).wait()

        fire(0, 0)

        def macro(gg, carry):
            for j in range(_NB):
                q = gg * _NB + j
                nb = (j + 1) % _NB

                @pl.when(q + 1 < _Q)
                def _():
                    @pl.when(q + 1 >= _NB)
                    def _():
                        wait_writeback(nb)

                    fire(q + 1, nb)

                wait_gathers(j)
                writeback(q, j)
            return carry

        lax.fori_loop(0, _Q // _NB, macro, 0)
        for b in range(_NB):
            wait_writeback(b)

    return body


_sc_gather = _make_sc_gather()


def kernel(text, seq_len, text_embed):
    text2d = text.reshape(_N // _G, _G)
    seqv = jnp.full((_L,), seq_len, dtype=jnp.int32)
    out = _sc_gather(text2d, seqv, text_embed)
    return out.reshape(_B, _T, _D)
